# Initial kernel scaffold; baseline (speedup 1.0000x reference)
import jax
import jax.numpy as jnp
from jax.experimental import pallas as pl


def kernel(nfeats, efeats, edge_index, Wni, Wnj, Wfij, Wn, bn, attn, be, Wmn, bmn, Wme, bme):
    raise NotImplementedError("write your pallas kernel here")


# speedup vs baseline: 18.5023x; 18.5023x over previous
import functools

import jax
import jax.numpy as jnp
from jax import lax
from jax.experimental import pallas as pl
from jax.experimental.pallas import tpu as pltpu
from jax.experimental.pallas import tpu_sc as plsc

_N = 10000
_E = 160000
_D = 128
_H = 2
_HD = _H * _D
_NC = 2
_NS = 16
_NW = _NC * _NS
_CH = 128
_NB = _E // _CH

_BN = 1000
_BE = 3200


def _node_proj_body(x_ref, wni_ref, wnj_ref, wn_ref, bn_ref,
                    fni_ref, fnj_ref, hsrc_ref):
    x = x_ref[...]
    fni_ref[...] = jnp.dot(x, wni_ref[...], preferred_element_type=jnp.float32)
    fnj_ref[...] = jnp.dot(x, wnj_ref[...], preferred_element_type=jnp.float32)
    hsrc_ref[...] = (jnp.dot(x, wn_ref[...], preferred_element_type=jnp.float32)
                     + bn_ref[...])


def _node_proj(x, wni, wnj, wn, bn):
    blk_w = pl.BlockSpec((_D, _HD), lambda i: (0, 0))
    return pl.pallas_call(
        _node_proj_body,
        grid=(_N // _BN,),
        in_specs=[
            pl.BlockSpec((_BN, _D), lambda i: (i, 0)),
            blk_w, blk_w, blk_w,
            pl.BlockSpec((1, _HD), lambda i: (0, 0)),
        ],
        out_specs=[pl.BlockSpec((_BN, _HD), lambda i: (i, 0))] * 3,
        out_shape=[jax.ShapeDtypeStruct((_N, _HD), jnp.float32)] * 3,
    )(x, wni, wnj, wn, bn)


def _gather_body(fni, fnj, hsrc, src, dst, g1, g2, g3, idx_s, idx_d, buf, sem):
    wid = lax.axis_index("s") * _NC + lax.axis_index("c")

    def step(k, carry):
        b = wid + k * _NW

        @pl.when(b < _NB)
        def _():
            off = b * _CH
            pltpu.sync_copy(src.at[pl.ds(off, _CH)], idx_s)
            pltpu.sync_copy(dst.at[pl.ds(off, _CH)], idx_d)
            pltpu.async_copy(fni.at[idx_s], buf, sem).wait()
            pltpu.sync_copy(buf, g1.at[pl.ds(off, _CH)])
            pltpu.async_copy(fnj.at[idx_d], buf, sem).wait()
            pltpu.sync_copy(buf, g2.at[pl.ds(off, _CH)])
            pltpu.async_copy(hsrc.at[idx_s], buf, sem).wait()
            pltpu.sync_copy(buf, g3.at[pl.ds(off, _CH)])

        return carry

    lax.fori_loop(0, (_NB + _NW - 1) // _NW, step, 0)


_gather = functools.partial(
    pl.kernel,
    mesh=plsc.VectorSubcoreMesh(core_axis_name="c", subcore_axis_name="s"),
    out_type=[jax.ShapeDtypeStruct((_E, _HD), jnp.float32)] * 3,
    scratch_types=[
        pltpu.VMEM((_CH,), jnp.int32),
        pltpu.VMEM((_CH,), jnp.int32),
        pltpu.VMEM((_CH, _HD), jnp.float32),
        pltpu.SemaphoreType.DMA,
    ],
)(_gather_body)


def _edge_body(g1_ref, g2_ref, g3_ref, e_ref, wf_ref, be_ref, attn_ref,
               wme_ref, bme_ref, msg_ref, at_ref, enew_ref):
    ew = jnp.dot(e_ref[...], wf_ref[...], preferred_element_type=jnp.float32)
    f = g1_ref[...] + g2_ref[...] + ew + be_ref[...]
    f = jnp.where(f >= 0.0, f, 0.01 * f)
    pa = f * attn_ref[...]
    a0 = jnp.exp(jnp.sum(pa[:, :_D], axis=1))
    a1 = jnp.exp(jnp.sum(pa[:, _D:], axis=1))
    at_ref[0, :] = a0
    at_ref[1, :] = a1
    g3 = g3_ref[...]
    msg_ref[0] = g3[:, :_D] * a0[:, None]
    msg_ref[1] = g3[:, _D:] * a1[:, None]
    en = jnp.dot(f, wme_ref[...], preferred_element_type=jnp.float32) + bme_ref[...]
    enew_ref[...] = jnp.where(en > 0.0, en, jnp.exp(en) - 1.0) + e_ref[...]


def _edge_stage(g1, g2, g3, e, wf, be, attn, wme, bme):
    blk_ehd = pl.BlockSpec((_BE, _HD), lambda i: (i, 0))
    return pl.pallas_call(
        _edge_body,
        grid=(_E // _BE,),
        in_specs=[
            blk_ehd, blk_ehd, blk_ehd,
            pl.BlockSpec((_BE, _D), lambda i: (i, 0)),
            pl.BlockSpec((_D, _HD), lambda i: (0, 0)),
            pl.BlockSpec((1, _HD), lambda i: (0, 0)),
            pl.BlockSpec((1, _HD), lambda i: (0, 0)),
            pl.BlockSpec((_HD, _D), lambda i: (0, 0)),
            pl.BlockSpec((1, _D), lambda i: (0, 0)),
        ],
        out_specs=[
            pl.BlockSpec((_H, _BE, _D), lambda i: (0, i, 0)),
            pl.BlockSpec((_H, _BE), lambda i: (0, i)),
            pl.BlockSpec((_BE, _D), lambda i: (i, 0)),
        ],
        out_shape=[
            jax.ShapeDtypeStruct((_H, _E, _D), jnp.float32),
            jax.ShapeDtypeStruct((_H, _E), jnp.float32),
            jax.ShapeDtypeStruct((_E, _D), jnp.float32),
        ],
    )(g1, g2, g3, e, wf, be, attn, wme, bme)


_ROWS = _N // _NS


def _scatter_body(msgT, atT, dst, z128, z1, hT,
                  idx_v, buf, aval, db, hb, h_acc, d_acc):
    core = lax.axis_index("c")
    sid = lax.axis_index("s")

    @pl.when(sid == 0)
    def _():
        pltpu.sync_copy(z128, h_acc)
        pltpu.sync_copy(z1, d_acc)

    plsc.subcore_barrier()

    def step(k, carry):
        b = sid + k * _NS

        @pl.when(b < _NB)
        def _():
            off = b * _CH
            pltpu.sync_copy(dst.at[pl.ds(off, _CH)], idx_v)
            pltpu.sync_copy(msgT.at[core, pl.ds(off, _CH)], buf)
            pltpu.sync_copy(buf, h_acc.at[idx_v], add=True)
            pltpu.sync_copy(atT.at[core, pl.ds(off, _CH)], aval)
            pltpu.sync_copy(aval, d_acc.at[idx_v], add=True)

        return carry

    lax.fori_loop(0, (_NB + _NS - 1) // _NS, step, 0)
    plsc.subcore_barrier()

    start = sid * 624
    nchunks = jnp.where(sid == _NS - 1, 40, 39)

    def norm_chunk(c, carry):
        base = start + c * 16
        pltpu.sync_copy(h_acc.at[pl.ds(base, 16)], hb)
        pltpu.sync_copy(d_acc.at[pl.ds(base, 16)], db)
        dv = db[...]
        inv = jnp.where(dv == 0.0, 1.0, 1.0 / dv)
        for r in range(16):
            for v in range(_D // 16):
                sl = pl.ds(v * 16, 16)
                hb[r, sl] = hb[r, sl] * inv[r]
        pltpu.sync_copy(hb, hT.at[core, pl.ds(base, 16)])
        return carry

    lax.fori_loop(0, nchunks, norm_chunk, 0)


_scatter = functools.partial(
    pl.kernel,
    mesh=plsc.VectorSubcoreMesh(core_axis_name="c", subcore_axis_name="s"),
    out_type=jax.ShapeDtypeStruct((_H, _N, _D), jnp.float32),
    scratch_types=[
        pltpu.VMEM((_CH,), jnp.int32),
        pltpu.VMEM((_CH, _D), jnp.float32),
        pltpu.VMEM((_CH,), jnp.float32),
        pltpu.VMEM((16,), jnp.float32),
        pltpu.VMEM((16, _D), jnp.float32),
        pltpu.VMEM_SHARED((_N, _D), jnp.float32),
        pltpu.VMEM_SHARED((_N,), jnp.float32),
    ],
)(_scatter_body)


def _node_update_body(h_ref, x_ref, wmn_ref, bmn_ref, xo_ref):
    hcat = jnp.concatenate([h_ref[0], h_ref[1]], axis=1)
    xn = jnp.dot(hcat, wmn_ref[...], preferred_element_type=jnp.float32) + bmn_ref[...]
    xo_ref[...] = jnp.where(xn > 0.0, xn, jnp.exp(xn) - 1.0) + x_ref[...]


def _node_update(hT, x, wmn, bmn):
    return pl.pallas_call(
        _node_update_body,
        grid=(_N // _BN,),
        in_specs=[
            pl.BlockSpec((_H, _BN, _D), lambda i: (0, i, 0)),
            pl.BlockSpec((_BN, _D), lambda i: (i, 0)),
            pl.BlockSpec((_HD, _D), lambda i: (0, 0)),
            pl.BlockSpec((1, _D), lambda i: (0, 0)),
        ],
        out_specs=pl.BlockSpec((_BN, _D), lambda i: (i, 0)),
        out_shape=jax.ShapeDtypeStruct((_N, _D), jnp.float32),
    )(hT, x, wmn, bmn)


def kernel(nfeats, efeats, edge_index, Wni, Wnj, Wfij, Wn, bn, attn, be,
           Wmn, bmn, Wme, bme):
    src = edge_index[0]
    dst = edge_index[1]
    z128 = jnp.zeros((_N, _D), jnp.float32)
    z1 = jnp.zeros((_N,), jnp.float32)
    x, e = nfeats, efeats
    for l in range(2):
        fni, fnj, hsrc = _node_proj(x, Wni[l], Wnj[l], Wn[l],
                                    bn[l].reshape(1, _HD))
        g1, g2, g3 = _gather(fni, fnj, hsrc, src, dst)
        msgT, atT, e = _edge_stage(g1, g2, g3, e, Wfij[l],
                                   be[l].reshape(1, _HD),
                                   attn[l].reshape(1, _HD),
                                   Wme[l], bme[l].reshape(1, _D))
        hT = _scatter(msgT, atT, dst, z128, z1)
        x = _node_update(hT, x, Wmn[l], bmn[l].reshape(1, _D))
    return x, e


# speedup vs baseline: 22.3052x; 1.2055x over previous
import functools

import jax
import jax.numpy as jnp
from jax import lax
from jax.experimental import pallas as pl
from jax.experimental.pallas import tpu as pltpu
from jax.experimental.pallas import tpu_sc as plsc

_N = 10000
_E = 160000
_D = 128
_H = 2
_HD = _H * _D
_NC = 2
_NS = 16
_NW = _NC * _NS
_CH = 128
_NB = _E // _CH

_BN = 1000
_BE = 3200


def _node_proj_body(x_ref, wni_ref, wnj_ref, wn_ref, bn_ref,
                    fni_ref, fnj_ref, hsrcT_ref):
    x = x_ref[...]
    fni_ref[...] = jnp.dot(x, wni_ref[...], preferred_element_type=jnp.float32)
    fnj_ref[...] = jnp.dot(x, wnj_ref[...], preferred_element_type=jnp.float32)
    hs = (jnp.dot(x, wn_ref[...], preferred_element_type=jnp.float32)
          + bn_ref[...])
    hsrcT_ref[0] = hs[:, :_D]
    hsrcT_ref[1] = hs[:, _D:]


def _node_proj(x, wni, wnj, wn, bn):
    blk_w = pl.BlockSpec((_D, _HD), lambda i: (0, 0))
    return pl.pallas_call(
        _node_proj_body,
        grid=(_N // _BN,),
        in_specs=[
            pl.BlockSpec((_BN, _D), lambda i: (i, 0)),
            blk_w, blk_w, blk_w,
            pl.BlockSpec((1, _HD), lambda i: (0, 0)),
        ],
        out_specs=[
            pl.BlockSpec((_BN, _HD), lambda i: (i, 0)),
            pl.BlockSpec((_BN, _HD), lambda i: (i, 0)),
            pl.BlockSpec((_H, _BN, _D), lambda i: (0, i, 0)),
        ],
        out_shape=[
            jax.ShapeDtypeStruct((_N, _HD), jnp.float32),
            jax.ShapeDtypeStruct((_N, _HD), jnp.float32),
            jax.ShapeDtypeStruct((_H, _N, _D), jnp.float32),
        ],
    )(x, wni, wnj, wn, bn)


_CHG = 80
_NBG = _E // _CHG
_NKG = (_NBG + _NW - 1) // _NW


def _gather_body(fni, fnj, src, dst, s_out,
                 is0, id0, is1, id1, bA0, bB0, bA1, bB1, semg):
    wid = lax.axis_index("s") * _NC + lax.axis_index("c")

    def issue(off, is_, id_, bA, bB):
        pltpu.sync_copy(src.at[pl.ds(off, _CHG)], is_)
        pltpu.sync_copy(dst.at[pl.ds(off, _CHG)], id_)
        pltpu.async_copy(fni.at[is_], bA, semg)
        pltpu.async_copy(fnj.at[id_], bB, semg)

    def finish(off, is_, id_, bA, bB):
        pltpu.make_async_copy(fni.at[is_], bA, semg).wait()
        pltpu.make_async_copy(fnj.at[id_], bB, semg).wait()

        def row(r, c):
            for v in range(_HD // 16):
                sl = pl.ds(v * 16, 16)
                bA[r, sl] = bA[r, sl] + bB[r, sl]
            return c

        lax.fori_loop(0, _CHG, row, 0)
        pltpu.sync_copy(bA, s_out.at[pl.ds(off, _CHG)])

    def step(k2, carry):
        b0 = wid + (2 * k2) * _NW
        b1 = wid + (2 * k2 + 1) * _NW

        @pl.when(b0 < _NBG)
        def _():
            issue(b0 * _CHG, is0, id0, bA0, bB0)

        @pl.when(b1 < _NBG)
        def _():
            issue(b1 * _CHG, is1, id1, bA1, bB1)

        @pl.when(b0 < _NBG)
        def _():
            finish(b0 * _CHG, is0, id0, bA0, bB0)

        @pl.when(b1 < _NBG)
        def _():
            finish(b1 * _CHG, is1, id1, bA1, bB1)

        return carry

    lax.fori_loop(0, (_NKG + 1) // 2, step, 0)


_gather = functools.partial(
    pl.kernel,
    mesh=plsc.VectorSubcoreMesh(core_axis_name="c", subcore_axis_name="s"),
    out_type=jax.ShapeDtypeStruct((_E, _HD), jnp.float32),
    scratch_types=[
        pltpu.VMEM((_CHG,), jnp.int32),
        pltpu.VMEM((_CHG,), jnp.int32),
        pltpu.VMEM((_CHG,), jnp.int32),
        pltpu.VMEM((_CHG,), jnp.int32),
        pltpu.VMEM((_CHG, _HD), jnp.float32),
        pltpu.VMEM((_CHG, _HD), jnp.float32),
        pltpu.VMEM((_CHG, _HD), jnp.float32),
        pltpu.VMEM((_CHG, _HD), jnp.float32),
        pltpu.SemaphoreType.DMA,
    ],
)(_gather_body)


def _edge_body(s_ref, e_ref, wf_ref, be_ref, attn_ref,
               wme_ref, bme_ref, at_ref, enew_ref):
    ew = jnp.dot(e_ref[...], wf_ref[...], preferred_element_type=jnp.float32)
    f = s_ref[...] + ew + be_ref[...]
    f = jnp.where(f >= 0.0, f, 0.01 * f)
    pa = f * attn_ref[...]
    at_ref[0, :] = jnp.exp(jnp.sum(pa[:, :_D], axis=1))
    at_ref[1, :] = jnp.exp(jnp.sum(pa[:, _D:], axis=1))
    en = jnp.dot(f, wme_ref[...], preferred_element_type=jnp.float32) + bme_ref[...]
    enew_ref[...] = jnp.where(en > 0.0, en, jnp.exp(en) - 1.0) + e_ref[...]


def _edge_stage(s, e, wf, be, attn, wme, bme):
    return pl.pallas_call(
        _edge_body,
        grid=(_E // _BE,),
        in_specs=[
            pl.BlockSpec((_BE, _HD), lambda i: (i, 0)),
            pl.BlockSpec((_BE, _D), lambda i: (i, 0)),
            pl.BlockSpec((_D, _HD), lambda i: (0, 0)),
            pl.BlockSpec((1, _HD), lambda i: (0, 0)),
            pl.BlockSpec((1, _HD), lambda i: (0, 0)),
            pl.BlockSpec((_HD, _D), lambda i: (0, 0)),
            pl.BlockSpec((1, _D), lambda i: (0, 0)),
        ],
        out_specs=[
            pl.BlockSpec((_H, _BE), lambda i: (0, i)),
            pl.BlockSpec((_BE, _D), lambda i: (i, 0)),
        ],
        out_shape=[
            jax.ShapeDtypeStruct((_H, _E), jnp.float32),
            jax.ShapeDtypeStruct((_E, _D), jnp.float32),
        ],
    )(s, e, wf, be, attn, wme, bme)


_ROWS = _N // _NS


def _scatter_body(hsrcT, atT, src, dst, z128, z1, hT,
                  idx_s, idx_d, buf, aval, db, hb, h_acc, d_acc, sem):
    core = lax.axis_index("c")
    sid = lax.axis_index("s")

    @pl.when(sid == 0)
    def _():
        pltpu.sync_copy(z128, h_acc)
        pltpu.sync_copy(z1, d_acc)

    plsc.subcore_barrier()

    def step(k, carry):
        b = sid + k * _NS

        @pl.when(b < _NB)
        def _():
            off = b * _CH
            pltpu.sync_copy(src.at[pl.ds(off, _CH)], idx_s)
            pltpu.sync_copy(dst.at[pl.ds(off, _CH)], idx_d)
            pltpu.async_copy(hsrcT.at[core].at[idx_s], buf, sem).wait()
            pltpu.sync_copy(atT.at[core, pl.ds(off, _CH)], aval)

            def scale_group(g, c):
                av = aval[pl.ds(g * 16, 16)]
                for r in range(16):
                    row = g * 16 + r
                    for v in range(_D // 16):
                        sl = pl.ds(v * 16, 16)
                        buf[row, sl] = buf[row, sl] * av[r]
                return c

            lax.fori_loop(0, _CH // 16, scale_group, 0)
            pltpu.sync_copy(buf, h_acc.at[idx_d], add=True)
            pltpu.sync_copy(aval, d_acc.at[idx_d], add=True)

        return carry

    lax.fori_loop(0, (_NB + _NS - 1) // _NS, step, 0)
    plsc.subcore_barrier()

    start = sid * 624
    nchunks = jnp.where(sid == _NS - 1, 40, 39)

    def norm_chunk(c, carry):
        base = start + c * 16
        pltpu.sync_copy(h_acc.at[pl.ds(base, 16)], hb)
        pltpu.sync_copy(d_acc.at[pl.ds(base, 16)], db)
        dv = db[...]
        inv = jnp.where(dv == 0.0, 1.0, 1.0 / dv)
        for r in range(16):
            for v in range(_D // 16):
                sl = pl.ds(v * 16, 16)
                hb[r, sl] = hb[r, sl] * inv[r]
        pltpu.sync_copy(hb, hT.at[core, pl.ds(base, 16)])
        return carry

    lax.fori_loop(0, nchunks, norm_chunk, 0)


_scatter = functools.partial(
    pl.kernel,
    mesh=plsc.VectorSubcoreMesh(core_axis_name="c", subcore_axis_name="s"),
    out_type=jax.ShapeDtypeStruct((_H, _N, _D), jnp.float32),
    scratch_types=[
        pltpu.VMEM((_CH,), jnp.int32),
        pltpu.VMEM((_CH,), jnp.int32),
        pltpu.VMEM((_CH, _D), jnp.float32),
        pltpu.VMEM((_CH,), jnp.float32),
        pltpu.VMEM((16,), jnp.float32),
        pltpu.VMEM((16, _D), jnp.float32),
        pltpu.VMEM_SHARED((_N, _D), jnp.float32),
        pltpu.VMEM_SHARED((_N,), jnp.float32),
        pltpu.SemaphoreType.DMA,
    ],
)(_scatter_body)


def _node_update_body(h_ref, x_ref, wmn_ref, bmn_ref, xo_ref):
    hcat = jnp.concatenate([h_ref[0], h_ref[1]], axis=1)
    xn = jnp.dot(hcat, wmn_ref[...], preferred_element_type=jnp.float32) + bmn_ref[...]
    xo_ref[...] = jnp.where(xn > 0.0, xn, jnp.exp(xn) - 1.0) + x_ref[...]


def _node_update(hT, x, wmn, bmn):
    return pl.pallas_call(
        _node_update_body,
        grid=(_N // _BN,),
        in_specs=[
            pl.BlockSpec((_H, _BN, _D), lambda i: (0, i, 0)),
            pl.BlockSpec((_BN, _D), lambda i: (i, 0)),
            pl.BlockSpec((_HD, _D), lambda i: (0, 0)),
            pl.BlockSpec((1, _D), lambda i: (0, 0)),
        ],
        out_specs=pl.BlockSpec((_BN, _D), lambda i: (i, 0)),
        out_shape=jax.ShapeDtypeStruct((_N, _D), jnp.float32),
    )(hT, x, wmn, bmn)


def kernel(nfeats, efeats, edge_index, Wni, Wnj, Wfij, Wn, bn, attn, be,
           Wmn, bmn, Wme, bme):
    src = edge_index[0]
    dst = edge_index[1]
    z128 = jnp.zeros((_N, _D), jnp.float32)
    z1 = jnp.zeros((_N,), jnp.float32)
    x, e = nfeats, efeats
    for l in range(2):
        fni, fnj, hsrcT = _node_proj(x, Wni[l], Wnj[l], Wn[l],
                                     bn[l].reshape(1, _HD))
        s = _gather(fni, fnj, src, dst)
        atT, e = _edge_stage(s, e, Wfij[l],
                             be[l].reshape(1, _HD),
                             attn[l].reshape(1, _HD),
                             Wme[l], bme[l].reshape(1, _D))
        hT = _scatter(hsrcT, atT, src, dst, z128, z1)
        x = _node_update(hT, x, Wmn[l], bmn[l].reshape(1, _D))
    return x, e


# speedup vs baseline: 25.8930x; 1.1609x over previous
import functools

import jax
import jax.numpy as jnp
from jax import lax
from jax.experimental import pallas as pl
from jax.experimental.pallas import tpu as pltpu
from jax.experimental.pallas import tpu_sc as plsc

_N = 10000
_E = 160000
_D = 128
_H = 2
_HD = _H * _D
_NC = 2
_NS = 16
_NW = _NC * _NS
_CH = 128
_NB = _E // _CH

_BN = 1000
_BE = 3200


def _node_proj_body(x_ref, wni_ref, wnj_ref, wn_ref, bn_ref,
                    fni_ref, fnj_ref, hsrcT_ref):
    x = x_ref[...]
    fni_ref[...] = jnp.dot(x, wni_ref[...], preferred_element_type=jnp.float32)
    fnj_ref[...] = jnp.dot(x, wnj_ref[...], preferred_element_type=jnp.float32)
    hs = (jnp.dot(x, wn_ref[...], preferred_element_type=jnp.float32)
          + bn_ref[...])
    hsrcT_ref[0] = hs[:, :_D]
    hsrcT_ref[1] = hs[:, _D:]


def _node_proj(x, wni, wnj, wn, bn):
    blk_w = pl.BlockSpec((_D, _HD), lambda i: (0, 0))
    return pl.pallas_call(
        _node_proj_body,
        grid=(_N // _BN,),
        in_specs=[
            pl.BlockSpec((_BN, _D), lambda i: (i, 0)),
            blk_w, blk_w, blk_w,
            pl.BlockSpec((1, _HD), lambda i: (0, 0)),
        ],
        out_specs=[
            pl.BlockSpec((_BN, _HD), lambda i: (i, 0)),
            pl.BlockSpec((_BN, _HD), lambda i: (i, 0)),
            pl.BlockSpec((_H, _BN, _D), lambda i: (0, i, 0)),
        ],
        out_shape=[
            jax.ShapeDtypeStruct((_N, _HD), jnp.float32),
            jax.ShapeDtypeStruct((_N, _HD), jnp.float32),
            jax.ShapeDtypeStruct((_H, _N, _D), jnp.float32),
        ],
    )(x, wni, wnj, wn, bn)


_CHG = 80
_NBG = _E // _CHG
_NKG = (_NBG + _NW - 1) // _NW


def _gather_body(fni, fnj, src, dst, s_out,
                 is0, id0, is1, id1, bA0, bB0, bA1, bB1, semg):
    wid = lax.axis_index("s") * _NC + lax.axis_index("c")

    def issue(off, is_, id_, bA, bB):
        pltpu.sync_copy(src.at[pl.ds(off, _CHG)], is_)
        pltpu.sync_copy(dst.at[pl.ds(off, _CHG)], id_)
        pltpu.async_copy(fni.at[is_], bA, semg)
        pltpu.async_copy(fnj.at[id_], bB, semg)

    def finish(off, is_, id_, bA, bB):
        pltpu.make_async_copy(fni.at[is_], bA, semg).wait()
        pltpu.make_async_copy(fnj.at[id_], bB, semg).wait()

        def row(r, c):
            for v in range(_HD // 16):
                sl = pl.ds(v * 16, 16)
                bA[r, sl] = bA[r, sl] + bB[r, sl]
            return c

        lax.fori_loop(0, _CHG, row, 0)
        pltpu.sync_copy(bA, s_out.at[pl.ds(off, _CHG)])

    def step(k2, carry):
        b0 = wid + (2 * k2) * _NW
        b1 = wid + (2 * k2 + 1) * _NW

        @pl.when(b0 < _NBG)
        def _():
            issue(b0 * _CHG, is0, id0, bA0, bB0)

        @pl.when(b1 < _NBG)
        def _():
            issue(b1 * _CHG, is1, id1, bA1, bB1)

        @pl.when(b0 < _NBG)
        def _():
            finish(b0 * _CHG, is0, id0, bA0, bB0)

        @pl.when(b1 < _NBG)
        def _():
            finish(b1 * _CHG, is1, id1, bA1, bB1)

        return carry

    lax.fori_loop(0, (_NKG + 1) // 2, step, 0)


_gather = functools.partial(
    pl.kernel,
    mesh=plsc.VectorSubcoreMesh(core_axis_name="c", subcore_axis_name="s"),
    out_type=jax.ShapeDtypeStruct((_E, _HD), jnp.float32),
    scratch_types=[
        pltpu.VMEM((_CHG,), jnp.int32),
        pltpu.VMEM((_CHG,), jnp.int32),
        pltpu.VMEM((_CHG,), jnp.int32),
        pltpu.VMEM((_CHG,), jnp.int32),
        pltpu.VMEM((_CHG, _HD), jnp.float32),
        pltpu.VMEM((_CHG, _HD), jnp.float32),
        pltpu.VMEM((_CHG, _HD), jnp.float32),
        pltpu.VMEM((_CHG, _HD), jnp.float32),
        pltpu.SemaphoreType.DMA,
    ],
)(_gather_body)


def _edge_body(s_ref, e_ref, wf_ref, be_ref, attn_ref,
               wme_ref, bme_ref, at_ref, enew_ref):
    ew = jnp.dot(e_ref[...], wf_ref[...], preferred_element_type=jnp.float32)
    f = s_ref[...] + ew + be_ref[...]
    f = jnp.where(f >= 0.0, f, 0.01 * f)
    pa = f * attn_ref[...]
    at_ref[0, :] = jnp.exp(jnp.sum(pa[:, :_D], axis=1))
    at_ref[1, :] = jnp.exp(jnp.sum(pa[:, _D:], axis=1))
    en = jnp.dot(f, wme_ref[...], preferred_element_type=jnp.float32) + bme_ref[...]
    enew_ref[...] = jnp.where(en > 0.0, en, jnp.exp(en) - 1.0) + e_ref[...]


def _edge_stage(s, e, wf, be, attn, wme, bme):
    return pl.pallas_call(
        _edge_body,
        grid=(_E // _BE,),
        in_specs=[
            pl.BlockSpec((_BE, _HD), lambda i: (i, 0)),
            pl.BlockSpec((_BE, _D), lambda i: (i, 0)),
            pl.BlockSpec((_D, _HD), lambda i: (0, 0)),
            pl.BlockSpec((1, _HD), lambda i: (0, 0)),
            pl.BlockSpec((1, _HD), lambda i: (0, 0)),
            pl.BlockSpec((_HD, _D), lambda i: (0, 0)),
            pl.BlockSpec((1, _D), lambda i: (0, 0)),
        ],
        out_specs=[
            pl.BlockSpec((_H, _BE), lambda i: (0, i)),
            pl.BlockSpec((_BE, _D), lambda i: (i, 0)),
        ],
        out_shape=[
            jax.ShapeDtypeStruct((_H, _E), jnp.float32),
            jax.ShapeDtypeStruct((_E, _D), jnp.float32),
        ],
    )(s, e, wf, be, attn, wme, bme)


_ROWS = _N // _NS


def _scatter_body(hsrcT, atT, src, dst, z128, z1, hT,
                  is0, id0, is1, id1, buf0, buf1, av0, av1,
                  db, hb, h_acc, d_acc, sem):
    core = lax.axis_index("c")
    sid = lax.axis_index("s")

    @pl.when(sid == 0)
    def _():
        pltpu.sync_copy(z128, h_acc)
        pltpu.sync_copy(z1, d_acc)

    plsc.subcore_barrier()

    def issue(b, is_, id_, buf, aval):
        off = b * _CH
        pltpu.sync_copy(src.at[pl.ds(off, _CH)], is_)
        pltpu.sync_copy(dst.at[pl.ds(off, _CH)], id_)
        pltpu.async_copy(hsrcT.at[core].at[is_], buf, sem)
        pltpu.async_copy(atT.at[core, pl.ds(off, _CH)], aval, sem)

    def finish(b, is_, id_, buf, aval):
        off = b * _CH
        pltpu.make_async_copy(hsrcT.at[core].at[is_], buf, sem).wait()
        pltpu.make_async_copy(atT.at[core, pl.ds(off, _CH)], aval, sem).wait()

        def scale_group(g, c):
            av = aval[pl.ds(g * 16, 16)]
            for r in range(16):
                row = g * 16 + r
                for v in range(_D // 16):
                    sl = pl.ds(v * 16, 16)
                    buf[row, sl] = buf[row, sl] * av[r]
            return c

        lax.fori_loop(0, _CH // 16, scale_group, 0)
        pltpu.sync_copy(buf, h_acc.at[id_], add=True)
        pltpu.sync_copy(aval, d_acc.at[id_], add=True)

    def step(k2, carry):
        b0 = sid + (2 * k2) * _NS
        b1 = sid + (2 * k2 + 1) * _NS

        @pl.when(b0 < _NB)
        def _():
            issue(b0, is0, id0, buf0, av0)

        @pl.when(b1 < _NB)
        def _():
            issue(b1, is1, id1, buf1, av1)

        @pl.when(b0 < _NB)
        def _():
            finish(b0, is0, id0, buf0, av0)

        @pl.when(b1 < _NB)
        def _():
            finish(b1, is1, id1, buf1, av1)

        return carry

    lax.fori_loop(0, (_NB + 2 * _NS - 1) // (2 * _NS), step, 0)
    plsc.subcore_barrier()

    start = sid * 624
    nchunks = jnp.where(sid == _NS - 1, 40, 39)

    def norm_chunk(c, carry):
        base = start + c * 16
        pltpu.sync_copy(h_acc.at[pl.ds(base, 16)], hb)
        pltpu.sync_copy(d_acc.at[pl.ds(base, 16)], db)
        dv = db[...]
        inv = jnp.where(dv == 0.0, 1.0, 1.0 / dv)
        for r in range(16):
            for v in range(_D // 16):
                sl = pl.ds(v * 16, 16)
                hb[r, sl] = hb[r, sl] * inv[r]
        pltpu.sync_copy(hb, hT.at[core, pl.ds(base, 16)])
        return carry

    lax.fori_loop(0, nchunks, norm_chunk, 0)


_scatter = functools.partial(
    pl.kernel,
    mesh=plsc.VectorSubcoreMesh(core_axis_name="c", subcore_axis_name="s"),
    out_type=jax.ShapeDtypeStruct((_H, _N, _D), jnp.float32),
    scratch_types=[
        pltpu.VMEM((_CH,), jnp.int32),
        pltpu.VMEM((_CH,), jnp.int32),
        pltpu.VMEM((_CH,), jnp.int32),
        pltpu.VMEM((_CH,), jnp.int32),
        pltpu.VMEM((_CH, _D), jnp.float32),
        pltpu.VMEM((_CH, _D), jnp.float32),
        pltpu.VMEM((_CH,), jnp.float32),
        pltpu.VMEM((_CH,), jnp.float32),
        pltpu.VMEM((16,), jnp.float32),
        pltpu.VMEM((16, _D), jnp.float32),
        pltpu.VMEM_SHARED((_N, _D), jnp.float32),
        pltpu.VMEM_SHARED((_N,), jnp.float32),
        pltpu.SemaphoreType.DMA,
    ],
)(_scatter_body)


def _node_update_body(h_ref, x_ref, wmn_ref, bmn_ref, xo_ref):
    hcat = jnp.concatenate([h_ref[0], h_ref[1]], axis=1)
    xn = jnp.dot(hcat, wmn_ref[...], preferred_element_type=jnp.float32) + bmn_ref[...]
    xo_ref[...] = jnp.where(xn > 0.0, xn, jnp.exp(xn) - 1.0) + x_ref[...]


def _node_update(hT, x, wmn, bmn):
    return pl.pallas_call(
        _node_update_body,
        grid=(_N // _BN,),
        in_specs=[
            pl.BlockSpec((_H, _BN, _D), lambda i: (0, i, 0)),
            pl.BlockSpec((_BN, _D), lambda i: (i, 0)),
            pl.BlockSpec((_HD, _D), lambda i: (0, 0)),
            pl.BlockSpec((1, _D), lambda i: (0, 0)),
        ],
        out_specs=pl.BlockSpec((_BN, _D), lambda i: (i, 0)),
        out_shape=jax.ShapeDtypeStruct((_N, _D), jnp.float32),
    )(hT, x, wmn, bmn)


def kernel(nfeats, efeats, edge_index, Wni, Wnj, Wfij, Wn, bn, attn, be,
           Wmn, bmn, Wme, bme):
    src = edge_index[0]
    dst = edge_index[1]
    z128 = jnp.zeros((_N, _D), jnp.float32)
    z1 = jnp.zeros((_N,), jnp.float32)
    x, e = nfeats, efeats
    for l in range(2):
        fni, fnj, hsrcT = _node_proj(x, Wni[l], Wnj[l], Wn[l],
                                     bn[l].reshape(1, _HD))
        s = _gather(fni, fnj, src, dst)
        atT, e = _edge_stage(s, e, Wfij[l],
                             be[l].reshape(1, _HD),
                             attn[l].reshape(1, _HD),
                             Wme[l], bme[l].reshape(1, _D))
        hT = _scatter(hsrcT, atT, src, dst, z128, z1)
        x = _node_update(hT, x, Wmn[l], bmn[l].reshape(1, _D))
    return x, e


# speedup vs baseline: 27.9928x; 1.0811x over previous
import functools

import jax
import jax.numpy as jnp
from jax import lax
from jax.experimental import pallas as pl
from jax.experimental.pallas import tpu as pltpu
from jax.experimental.pallas import tpu_sc as plsc

_N = 10000
_E = 160000
_D = 128
_H = 2
_HD = _H * _D
_NC = 2
_NS = 16
_NW = _NC * _NS
_CH = 128
_NB = _E // _CH

_BN = 1000
_BE = 3200


def _node_proj_body(x_ref, wni_ref, wnj_ref, wn_ref, bn_ref,
                    fni_ref, fnj_ref, hsrcT_ref):
    x = x_ref[...]
    fni_ref[...] = jnp.dot(x, wni_ref[...], preferred_element_type=jnp.float32)
    fnj_ref[...] = jnp.dot(x, wnj_ref[...], preferred_element_type=jnp.float32)
    hs = (jnp.dot(x, wn_ref[...], preferred_element_type=jnp.float32)
          + bn_ref[...])
    hsrcT_ref[0] = hs[:, :_D]
    hsrcT_ref[1] = hs[:, _D:]


def _node_proj(x, wni, wnj, wn, bn):
    blk_w = pl.BlockSpec((_D, _HD), lambda i: (0, 0))
    return pl.pallas_call(
        _node_proj_body,
        grid=(_N // _BN,),
        in_specs=[
            pl.BlockSpec((_BN, _D), lambda i: (i, 0)),
            blk_w, blk_w, blk_w,
            pl.BlockSpec((1, _HD), lambda i: (0, 0)),
        ],
        out_specs=[
            pl.BlockSpec((_BN, _HD), lambda i: (i, 0)),
            pl.BlockSpec((_BN, _HD), lambda i: (i, 0)),
            pl.BlockSpec((_H, _BN, _D), lambda i: (0, i, 0)),
        ],
        out_shape=[
            jax.ShapeDtypeStruct((_N, _HD), jnp.float32),
            jax.ShapeDtypeStruct((_N, _HD), jnp.float32),
            jax.ShapeDtypeStruct((_H, _N, _D), jnp.float32),
        ],
    )(x, wni, wnj, wn, bn)


_CHG = 80
_NBG = _E // _CHG
_NKG = (_NBG + _NW - 1) // _NW


def _gather_body(fni, fnj, src, dst, s_out,
                 is0, id0, is1, id1, bA0, bB0, bA1, bB1, semg):
    wid = lax.axis_index("s") * _NC + lax.axis_index("c")

    def issue(off, is_, id_, bA, bB):
        pltpu.sync_copy(src.at[pl.ds(off, _CHG)], is_)
        pltpu.sync_copy(dst.at[pl.ds(off, _CHG)], id_)
        pltpu.async_copy(fni.at[is_], bA, semg)
        pltpu.async_copy(fnj.at[id_], bB, semg)

    def finish(off, is_, id_, bA, bB):
        pltpu.make_async_copy(fni.at[is_], bA, semg).wait()
        pltpu.make_async_copy(fnj.at[id_], bB, semg).wait()

        def row(r, c):
            for v in range(_HD // 16):
                sl = pl.ds(v * 16, 16)
                bA[r, sl] = bA[r, sl] + bB[r, sl]
            return c

        lax.fori_loop(0, _CHG, row, 0)
        pltpu.sync_copy(bA, s_out.at[pl.ds(off, _CHG)])

    @pl.when(wid < _NBG)
    def _():
        issue(wid * _CHG, is0, id0, bA0, bB0)

    def step(k, carry):
        b = wid + k * _NW
        bn = wid + (k + 1) * _NW
        even = (k % 2) == 0

        @pl.when(jnp.logical_and(bn < _NBG, even))
        def _():
            issue(bn * _CHG, is1, id1, bA1, bB1)

        @pl.when(jnp.logical_and(bn < _NBG, jnp.logical_not(even)))
        def _():
            issue(bn * _CHG, is0, id0, bA0, bB0)

        @pl.when(jnp.logical_and(b < _NBG, even))
        def _():
            finish(b * _CHG, is0, id0, bA0, bB0)

        @pl.when(jnp.logical_and(b < _NBG, jnp.logical_not(even)))
        def _():
            finish(b * _CHG, is1, id1, bA1, bB1)

        return carry

    lax.fori_loop(0, _NKG, step, 0)


_gather = functools.partial(
    pl.kernel,
    mesh=plsc.VectorSubcoreMesh(core_axis_name="c", subcore_axis_name="s"),
    out_type=jax.ShapeDtypeStruct((_E, _HD), jnp.float32),
    scratch_types=[
        pltpu.VMEM((_CHG,), jnp.int32),
        pltpu.VMEM((_CHG,), jnp.int32),
        pltpu.VMEM((_CHG,), jnp.int32),
        pltpu.VMEM((_CHG,), jnp.int32),
        pltpu.VMEM((_CHG, _HD), jnp.float32),
        pltpu.VMEM((_CHG, _HD), jnp.float32),
        pltpu.VMEM((_CHG, _HD), jnp.float32),
        pltpu.VMEM((_CHG, _HD), jnp.float32),
        pltpu.SemaphoreType.DMA,
    ],
)(_gather_body)


def _edge_body(s_ref, e_ref, wf_ref, be_ref, attn_ref,
               wme_ref, bme_ref, at_ref, enew_ref):
    ew = jnp.dot(e_ref[...], wf_ref[...], preferred_element_type=jnp.float32)
    f = s_ref[...] + ew + be_ref[...]
    f = jnp.where(f >= 0.0, f, 0.01 * f)
    pa = f * attn_ref[...]
    at_ref[0, :] = jnp.exp(jnp.sum(pa[:, :_D], axis=1))
    at_ref[1, :] = jnp.exp(jnp.sum(pa[:, _D:], axis=1))
    en = jnp.dot(f, wme_ref[...], preferred_element_type=jnp.float32) + bme_ref[...]
    enew_ref[...] = jnp.where(en > 0.0, en, jnp.exp(en) - 1.0) + e_ref[...]


def _edge_stage(s, e, wf, be, attn, wme, bme):
    return pl.pallas_call(
        _edge_body,
        grid=(_E // _BE,),
        in_specs=[
            pl.BlockSpec((_BE, _HD), lambda i: (i, 0)),
            pl.BlockSpec((_BE, _D), lambda i: (i, 0)),
            pl.BlockSpec((_D, _HD), lambda i: (0, 0)),
            pl.BlockSpec((1, _HD), lambda i: (0, 0)),
            pl.BlockSpec((1, _HD), lambda i: (0, 0)),
            pl.BlockSpec((_HD, _D), lambda i: (0, 0)),
            pl.BlockSpec((1, _D), lambda i: (0, 0)),
        ],
        out_specs=[
            pl.BlockSpec((_H, _BE), lambda i: (0, i)),
            pl.BlockSpec((_BE, _D), lambda i: (i, 0)),
        ],
        out_shape=[
            jax.ShapeDtypeStruct((_H, _E), jnp.float32),
            jax.ShapeDtypeStruct((_E, _D), jnp.float32),
        ],
    )(s, e, wf, be, attn, wme, bme)


_ROWS = _N // _NS


def _scatter_body(hsrcT, atT, src, dst, z128, z1, hT,
                  is0, id0, is1, id1, buf0, buf1, av0, av1,
                  db, hb, h_acc, d_acc, sem):
    core = lax.axis_index("c")
    sid = lax.axis_index("s")

    @pl.when(sid == 0)
    def _():
        pltpu.sync_copy(z128, h_acc)
        pltpu.sync_copy(z1, d_acc)

    plsc.subcore_barrier()

    def issue(b, is_, id_, buf, aval):
        off = b * _CH
        pltpu.sync_copy(src.at[pl.ds(off, _CH)], is_)
        pltpu.sync_copy(dst.at[pl.ds(off, _CH)], id_)
        pltpu.async_copy(hsrcT.at[core].at[is_], buf, sem)
        pltpu.async_copy(atT.at[core, pl.ds(off, _CH)], aval, sem)

    def finish(b, is_, id_, buf, aval):
        off = b * _CH
        pltpu.make_async_copy(hsrcT.at[core].at[is_], buf, sem).wait()
        pltpu.make_async_copy(atT.at[core, pl.ds(off, _CH)], aval, sem).wait()

        def scale_group(g, c):
            av = aval[pl.ds(g * 16, 16)]
            for r in range(16):
                row = g * 16 + r
                for v in range(_D // 16):
                    sl = pl.ds(v * 16, 16)
                    buf[row, sl] = buf[row, sl] * av[r]
            return c

        lax.fori_loop(0, _CH // 16, scale_group, 0)
        pltpu.sync_copy(buf, h_acc.at[id_], add=True)
        pltpu.sync_copy(aval, d_acc.at[id_], add=True)

    @pl.when(sid < _NB)
    def _():
        issue(sid, is0, id0, buf0, av0)

    def step(k, carry):
        b = sid + k * _NS
        bn = sid + (k + 1) * _NS
        even = (k % 2) == 0

        @pl.when(jnp.logical_and(bn < _NB, even))
        def _():
            issue(bn, is1, id1, buf1, av1)

        @pl.when(jnp.logical_and(bn < _NB, jnp.logical_not(even)))
        def _():
            issue(bn, is0, id0, buf0, av0)

        @pl.when(jnp.logical_and(b < _NB, even))
        def _():
            finish(b, is0, id0, buf0, av0)

        @pl.when(jnp.logical_and(b < _NB, jnp.logical_not(even)))
        def _():
            finish(b, is1, id1, buf1, av1)

        return carry

    lax.fori_loop(0, (_NB + _NS - 1) // _NS, step, 0)
    plsc.subcore_barrier()

    start = sid * 624
    nchunks = jnp.where(sid == _NS - 1, 40, 39)

    def norm_chunk(c, carry):
        base = start + c * 16
        pltpu.sync_copy(h_acc.at[pl.ds(base, 16)], hb)
        pltpu.sync_copy(d_acc.at[pl.ds(base, 16)], db)
        dv = db[...]
        inv = jnp.where(dv == 0.0, 1.0, 1.0 / dv)
        for r in range(16):
            for v in range(_D // 16):
                sl = pl.ds(v * 16, 16)
                hb[r, sl] = hb[r, sl] * inv[r]
        pltpu.sync_copy(hb, hT.at[core, pl.ds(base, 16)])
        return carry

    lax.fori_loop(0, nchunks, norm_chunk, 0)


_scatter = functools.partial(
    pl.kernel,
    mesh=plsc.VectorSubcoreMesh(core_axis_name="c", subcore_axis_name="s"),
    out_type=jax.ShapeDtypeStruct((_H, _N, _D), jnp.float32),
    scratch_types=[
        pltpu.VMEM((_CH,), jnp.int32),
        pltpu.VMEM((_CH,), jnp.int32),
        pltpu.VMEM((_CH,), jnp.int32),
        pltpu.VMEM((_CH,), jnp.int32),
        pltpu.VMEM((_CH, _D), jnp.float32),
        pltpu.VMEM((_CH, _D), jnp.float32),
        pltpu.VMEM((_CH,), jnp.float32),
        pltpu.VMEM((_CH,), jnp.float32),
        pltpu.VMEM((16,), jnp.float32),
        pltpu.VMEM((16, _D), jnp.float32),
        pltpu.VMEM_SHARED((_N, _D), jnp.float32),
        pltpu.VMEM_SHARED((_N,), jnp.float32),
        pltpu.SemaphoreType.DMA,
    ],
)(_scatter_body)


def _node_update_body(h_ref, x_ref, wmn_ref, bmn_ref, xo_ref):
    hcat = jnp.concatenate([h_ref[0], h_ref[1]], axis=1)
    xn = jnp.dot(hcat, wmn_ref[...], preferred_element_type=jnp.float32) + bmn_ref[...]
    xo_ref[...] = jnp.where(xn > 0.0, xn, jnp.exp(xn) - 1.0) + x_ref[...]


def _node_update(hT, x, wmn, bmn):
    return pl.pallas_call(
        _node_update_body,
        grid=(_N // _BN,),
        in_specs=[
            pl.BlockSpec((_H, _BN, _D), lambda i: (0, i, 0)),
            pl.BlockSpec((_BN, _D), lambda i: (i, 0)),
            pl.BlockSpec((_HD, _D), lambda i: (0, 0)),
            pl.BlockSpec((1, _D), lambda i: (0, 0)),
        ],
        out_specs=pl.BlockSpec((_BN, _D), lambda i: (i, 0)),
        out_shape=jax.ShapeDtypeStruct((_N, _D), jnp.float32),
    )(hT, x, wmn, bmn)


def kernel(nfeats, efeats, edge_index, Wni, Wnj, Wfij, Wn, bn, attn, be,
           Wmn, bmn, Wme, bme):
    src = edge_index[0]
    dst = edge_index[1]
    z128 = jnp.zeros((_N, _D), jnp.float32)
    z1 = jnp.zeros((_N,), jnp.float32)
    x, e = nfeats, efeats
    for l in range(2):
        fni, fnj, hsrcT = _node_proj(x, Wni[l], Wnj[l], Wn[l],
                                     bn[l].reshape(1, _HD))
        s = _gather(fni, fnj, src, dst)
        atT, e = _edge_stage(s, e, Wfij[l],
                             be[l].reshape(1, _HD),
                             attn[l].reshape(1, _HD),
                             Wme[l], bme[l].reshape(1, _D))
        hT = _scatter(hsrcT, atT, src, dst, z128, z1)
        x = _node_update(hT, x, Wmn[l], bmn[l].reshape(1, _D))
    return x, e
